# baseline (speedup 1.0000x reference)
import functools

import jax
import jax.numpy as jnp
from jax import lax
from jax.experimental import pallas as pl
from jax.experimental.pallas import tpu as pltpu
from jax.experimental.pallas import tpu_sc as plsc

_FIELD_SIZES = (1000000, 3, 100, 50, 1000000, 5000, 300)
_NF = 7
_OFFS = (0, 1000000, 1000003, 1000103, 1000153, 2000153, 2005153)
_TOTAL = 2005453

_N = 16384
_K = 16
_L = 16
_NC = 2
_NS = 16
_NW = _NC * _NS
_RPW = _N // _NW
_NBLK = _RPW // _L

_BIG = (0, 4)
_CACHED = (1, 2, 3, 6, 5)
_TSTART = {f: (_OFFS[f] // 8) * 8 for f in _CACHED}
_TREM = {f: _OFFS[f] - _TSTART[f] for f in _CACHED}
_TB = {1: 0, 2: 16, 3: 128, 6: 192, 5: 496}
_TCOPY = {1: 16, 2: 112, 3: 64, 6: 301, 5: 5008}
_TROWS = 5504

_DN = lax.GatherDimensionNumbers(offset_dims=(), collapsed_slice_dims=(0,),
                                 start_index_map=(0,))


def _perm(v, table):
    return lax.gather(v, table[:, None], _DN, slice_sizes=(1,),
                      mode=lax.GatherScatterMode.PROMISE_IN_BOUNDS)


def _fm_body(f0, f1, f2, f3, f4, f5, f6, w_so, w_lin, out,
             idx_v, bidx_v, so_tab, lin_tab, so_v, linbig_v, out_v, sem):
    fields = (f0, f1, f2, f3, f4, f5, f6)
    wid = lax.axis_index("s") * _NC + lax.axis_index("c")
    base = wid * _RPW
    lanes = lax.iota(jnp.int32, _L)

    copies = []
    for f in range(_NF):
        copies.append(pltpu.async_copy(fields[f].at[pl.ds(base, _RPW)],
                                       idx_v.at[f], sem))

    for f in _CACHED:
        copies.append(pltpu.async_copy(
            w_so.at[pl.ds(_TSTART[f], _TCOPY[f])],
            so_tab.at[pl.ds(_TB[f], _TCOPY[f])], sem))
        copies.append(pltpu.async_copy(
            w_lin.at[pl.ds(_TSTART[f], _TCOPY[f])],
            lin_tab.at[pl.ds(_TB[f], _TCOPY[f])], sem))
    for cp in copies:
        cp.wait()

    def _bld(i, c_):
        sl = pl.ds(i * _L, _L)
        bidx_v[sl] = idx_v[0, sl]
        bidx_v[pl.ds(_RPW + i * _L, _L)] = idx_v[4, sl] + _OFFS[4]
        return c_
    lax.fori_loop(0, _RPW // _L, _bld, 0)

    g1 = pltpu.async_copy(w_so.at[bidx_v.at[pl.ds(0, _RPW)]],
                          so_v.at[pl.ds(0, _RPW)], sem)
    g2 = pltpu.async_copy(w_so.at[bidx_v.at[pl.ds(_RPW, _RPW)]],
                          so_v.at[pl.ds(_RPW, _RPW)], sem)
    g3 = pltpu.async_copy(w_lin.at[bidx_v], linbig_v, sem)
    g1.wait(); g2.wait(); g3.wait()

    btabs = {h: (lanes ^ h, (lanes & h) == 0) for h in (8, 4, 2, 1)}
    lane0 = lanes == 0

    def _blk(i, c_):
        row0 = i * _L
        sl = pl.ds(row0, _L)
        tiv = [idx_v[f, sl] for f in _CACHED]
        vecs = []
        for r in range(_L):
            b = row0 + r
            s = so_v[b, :]
            t = s * s
            v = so_v[_RPW + b, :]
            s = s + v
            t = t + v * v
            linacc = None
            for q, f in enumerate(_CACHED):
                xi = tiv[q][r]
                trow = (_TB[f] + _TREM[f]) + xi
                v = so_tab[trow, :]
                s = s + v
                t = t + v * v
                lv = lin_tab[pl.ds(trow, _L)]
                linacc = lv if linacc is None else linacc + lv
            d = s * s - t + jnp.where(lane0, 2.0 * linacc, 0.0)
            vecs.append(d)
        h = 8
        while len(vecs) > 1:
            n = len(vecs) // 2
            partner, mask = btabs[h]
            vecs = [jnp.where(mask,
                              vecs[q] + _perm(vecs[q], partner),
                              _perm(vecs[q + n] + _perm(vecs[q + n], partner),
                                    partner))
                    for q in range(n)]
            h //= 2
        linsum = linbig_v[sl] + linbig_v[pl.ds(_RPW + row0, _L)]
        out_v[sl] = linsum + 0.5 * vecs[0]
        return c_
    lax.fori_loop(0, _NBLK, _blk, 0)

    pltpu.sync_copy(out_v, out.at[pl.ds(base, _RPW)])


_fm_sc = functools.partial(
    pl.kernel,
    out_type=jax.ShapeDtypeStruct((_N,), jnp.float32),
    mesh=plsc.VectorSubcoreMesh(core_axis_name="c", subcore_axis_name="s",
                                num_cores=_NC, num_subcores=_NS),
    compiler_params=pltpu.CompilerParams(use_tc_tiling_on_sc=False),
    scratch_types=[
        pltpu.VMEM((_NF, _RPW), jnp.int32),
        pltpu.VMEM((2 * _RPW,), jnp.int32),
        pltpu.VMEM((_TROWS, _K), jnp.float32),
        pltpu.VMEM((_TROWS + _L,), jnp.float32),
        pltpu.VMEM((2 * _RPW, _K), jnp.float32),
        pltpu.VMEM((2 * _RPW,), jnp.float32),
        pltpu.VMEM((_RPW,), jnp.float32),
        pltpu.SemaphoreType.DMA,
    ],
)(_fm_body)


def kernel(user_id, user_gender, user_occupation, user_address, product_id,
           product_store_id, product_category_id, W_lin, W_so, bias):
    out = _fm_sc(user_id, user_gender, user_occupation, user_address,
                 product_id, product_store_id, product_category_id,
                 W_so, W_lin.reshape(-1))
    return out + bias


# speedup vs baseline: 1.0024x; 1.0024x over previous
import functools

import jax
import jax.numpy as jnp
from jax import lax
from jax.experimental import pallas as pl
from jax.experimental.pallas import tpu as pltpu
from jax.experimental.pallas import tpu_sc as plsc

_FIELD_SIZES = (1000000, 3, 100, 50, 1000000, 5000, 300)
_NF = 7
_OFFS = (0, 1000000, 1000003, 1000103, 1000153, 2000153, 2005153)
_TOTAL = 2005453

_N = 16384
_K = 16
_L = 16
_NC = 2
_NS = 16
_NW = _NC * _NS
_RPW = _N // _NW
_NBLK = _RPW // _L

_BIG = (0, 4)
_CACHED = (1, 2, 3, 6, 5)
_TSTART = {f: (_OFFS[f] // 8) * 8 for f in _CACHED}
_TREM = {f: _OFFS[f] - _TSTART[f] for f in _CACHED}
_TB = {1: 0, 2: 16, 3: 128, 6: 192, 5: 496}
_TCOPY = {1: 16, 2: 112, 3: 64, 6: 301, 5: 5008}
_TROWS = 5504

_DN = lax.GatherDimensionNumbers(offset_dims=(), collapsed_slice_dims=(0,),
                                 start_index_map=(0,))


def _perm(v, table):
    return lax.gather(v, table[:, None], _DN, slice_sizes=(1,),
                      mode=lax.GatherScatterMode.PROMISE_IN_BOUNDS)


def _fm_body(f0, f1, f2, f3, f4, f5, f6, w_so, w_lin, out,
             idx_v, bidx_v, so_tab, lin_tab, so_v, linbig_v, out_v, sem):
    fields = (f0, f1, f2, f3, f4, f5, f6)
    wid = lax.axis_index("s") * _NC + lax.axis_index("c")
    base = wid * _RPW
    lanes = lax.iota(jnp.int32, _L)

    copies = []
    for f in range(_NF):
        copies.append(pltpu.async_copy(fields[f].at[pl.ds(base, _RPW)],
                                       idx_v.at[f], sem))

    for f in _CACHED:
        copies.append(pltpu.async_copy(
            w_so.at[pl.ds(_TSTART[f], _TCOPY[f])],
            so_tab.at[pl.ds(_TB[f], _TCOPY[f])], sem))
        copies.append(pltpu.async_copy(
            w_lin.at[pl.ds(_TSTART[f], _TCOPY[f])],
            lin_tab.at[pl.ds(_TB[f], _TCOPY[f])], sem))
    for cp in copies:
        cp.wait()

    def _bld(i, c_):
        sl = pl.ds(i * _L, _L)
        bidx_v[sl] = idx_v[0, sl]
        bidx_v[pl.ds(_RPW + i * _L, _L)] = idx_v[4, sl] + _OFFS[4]
        return c_
    lax.fori_loop(0, _RPW // _L, _bld, 0)

    g1 = pltpu.async_copy(w_so.at[bidx_v.at[pl.ds(0, _RPW)]],
                          so_v.at[pl.ds(0, _RPW)], sem)
    g2 = pltpu.async_copy(w_so.at[bidx_v.at[pl.ds(_RPW, _RPW)]],
                          so_v.at[pl.ds(_RPW, _RPW)], sem)
    g3 = pltpu.async_copy(w_lin.at[bidx_v], linbig_v, sem)
    g1.wait(); g2.wait(); g3.wait()

    btabs = {h: (lanes ^ h, (lanes & h) == 0) for h in (8, 4, 2, 1)}
    lane0 = lanes == 0

    def _blk(i, c_):
        row0 = i * _L
        sl = pl.ds(row0, _L)
        tiv = [idx_v[f, sl] for f in _CACHED]
        vecs = []
        for r in range(_L):
            b = row0 + r
            s = so_v[b, :]
            t = s * s
            v = so_v[_RPW + b, :]
            s = s + v
            t = t + v * v
            linacc = None
            for q, f in enumerate(_CACHED):
                xi = tiv[q][r]
                trow = (_TB[f] + _TREM[f]) + xi
                v = so_tab[trow, :]
                s = s + v
                t = t + v * v
                lv = lin_tab[pl.ds(trow, _L)]
                linacc = lv if linacc is None else linacc + lv
            d = s * s - t + jnp.where(lane0, 2.0 * linacc, 0.0)
            vecs.append(d)
        h = 8
        while len(vecs) > 1:
            n = len(vecs) // 2
            partner, mask = btabs[h]
            vecs = [jnp.where(mask,
                              vecs[q] + _perm(vecs[q], partner),
                              _perm(vecs[q + n] + _perm(vecs[q + n], partner),
                                    partner))
                    for q in range(n)]
            h //= 2
        linsum = linbig_v[sl] + linbig_v[pl.ds(_RPW + row0, _L)]
        out_v[sl] = linsum + 0.5 * vecs[0]
        return c_
    lax.fori_loop(0, _NBLK, _blk, 0)

    pltpu.sync_copy(out_v, out.at[pl.ds(base, _RPW)])


_fm_sc = functools.partial(
    pl.kernel,
    out_type=jax.ShapeDtypeStruct((_N,), jnp.float32),
    mesh=plsc.VectorSubcoreMesh(core_axis_name="c", subcore_axis_name="s",
                                num_cores=_NC, num_subcores=_NS),
    compiler_params=pltpu.CompilerParams(use_tc_tiling_on_sc=False),
    scratch_types=[
        pltpu.VMEM((_NF, _RPW), jnp.int32),
        pltpu.VMEM((2 * _RPW,), jnp.int32),
        pltpu.VMEM((_TROWS, _K), jnp.float32),
        pltpu.VMEM((_TROWS + _L,), jnp.float32),
        pltpu.VMEM((2 * _RPW, _K), jnp.float32),
        pltpu.VMEM((2 * _RPW,), jnp.float32),
        pltpu.VMEM((_RPW,), jnp.float32),
        pltpu.SemaphoreType.DMA,
    ],
)(_fm_body)


def kernel(user_id, user_gender, user_occupation, user_address, product_id,
           product_store_id, product_category_id, W_lin, W_so, bias):
    out = _fm_sc(user_id, user_gender, user_occupation, user_address,
                 product_id, product_store_id, product_category_id,
                 W_so, W_lin.T.reshape(-1))
    return out + bias
